# baseline (speedup 1.0000x reference)
import jax
import jax.numpy as jnp
from jax.experimental import pallas as pl
from jax.experimental.pallas import tpu as pltpu

_N = 10000
_D = 128
_K = 10
_BM = 200
_NS = 4
_NCHUNK = _N // _BM


def _fused(x_ref, A_ref, W1_ref, b_ref, W2_ref, cl_ref,
           h1_ref, sim_ref, bufs, z_ref, support_ref, cln_ref, sems):
    i = pl.program_id(0)

    def _copy(c, slot):
        return pltpu.make_async_copy(
            A_ref.at[pl.ds(c * _BM, _BM), :],
            bufs.at[slot],
            sems.at[slot],
        )

    @pl.when(i == 0)
    def _prelude():
        for s in range(_NS - 1):
            _copy(s, s).start()
        z = jnp.dot(x_ref[...], W1_ref[...],
                    preferred_element_type=jnp.float32) + b_ref[...]
        z_ref[...] = z
        support_ref[...] = jnp.dot(z, W2_ref[...],
                                   preferred_element_type=jnp.float32
                                   ).astype(jnp.bfloat16)
        cl = cl_ref[...]
        c_norm = jnp.sqrt(jnp.sum(cl * cl, axis=1, keepdims=True))
        cln_ref[...] = cl / jnp.maximum(c_norm, 1e-8)

    slot = jax.lax.rem(i, _NS)
    _copy(i, slot).wait()

    nxt = i + _NS - 1

    @pl.when(nxt < _NCHUNK)
    def _issue():
        _copy(nxt, jax.lax.rem(nxt, _NS)).start()

    h1_ref[pl.ds(i * _BM, _BM), :] = jnp.tanh(
        jnp.dot(bufs[slot].astype(jnp.bfloat16), support_ref[...],
                preferred_element_type=jnp.float32))

    z_blk = z_ref[pl.ds(i * _BM, _BM), :]
    num = jax.lax.dot_general(z_blk, cln_ref[...], (((1,), (1,)), ((), ())),
                              preferred_element_type=jnp.float32)
    z_norm = jnp.sqrt(jnp.sum(z_blk * z_blk, axis=1, keepdims=True))
    e = jnp.exp(num / jnp.maximum(z_norm, 1e-8))
    sim_ref[...] = e / jnp.sum(e, axis=1, keepdims=True)


def kernel(seq1, adj, W_ae1, b_ae1, W_gcn, cluster):
    x = seq1[0]
    A = adj[0]
    b2 = b_ae1.reshape(1, _D)
    grid = (_NCHUNK,)
    h1, sim = pl.pallas_call(
        _fused,
        grid=grid,
        in_specs=[
            pl.BlockSpec((_N, _D), lambda i: (0, 0)),
            pl.BlockSpec(memory_space=pl.ANY),
            pl.BlockSpec((_D, _D), lambda i: (0, 0)),
            pl.BlockSpec((1, _D), lambda i: (0, 0)),
            pl.BlockSpec((_D, _D), lambda i: (0, 0)),
            pl.BlockSpec((_K, _D), lambda i: (0, 0)),
        ],
        out_specs=[
            pl.BlockSpec((_N, _D), lambda i: (0, 0)),
            pl.BlockSpec((_BM, _K), lambda i: (i, 0)),
        ],
        out_shape=[
            jax.ShapeDtypeStruct((_N, _D), jnp.float32),
            jax.ShapeDtypeStruct((_N, _K), jnp.float32),
        ],
        scratch_shapes=[
            pltpu.VMEM((_NS, _BM, _N), jnp.float32),
            pltpu.VMEM((_N, _D), jnp.float32),
            pltpu.VMEM((_N, _D), jnp.bfloat16),
            pltpu.VMEM((_K, _D), jnp.float32),
            pltpu.SemaphoreType.DMA((_NS,)),
        ],
        compiler_params=pltpu.CompilerParams(
            dimension_semantics=("arbitrary",),
        ),
    )(x, A, W_ae1, b2, W_gcn, cluster)
    return (h1, sim)


# speedup vs baseline: 1.0056x; 1.0056x over previous
import jax
import jax.numpy as jnp
from jax.experimental import pallas as pl
from jax.experimental.pallas import tpu as pltpu

_N = 10000
_D = 128
_K = 10
_BM = 200
_HALVES = ((0, 104), (104, 96))
_NS = 4
_NCHUNK = _N // _BM


def _fused(x_ref, A_ref, W1_ref, b_ref, W2_ref, cl_ref,
           h1_ref, sim_ref, bufs, z_ref, support_ref, cln_ref, sems):
    i = pl.program_id(0)

    def _copies(c, slot):
        return [
            pltpu.make_async_copy(
                A_ref.at[pl.ds(c * _BM + off, rows), :],
                bufs.at[slot, pl.ds(off, rows)],
                sems.at[slot, h],
            )
            for h, (off, rows) in enumerate(_HALVES)
        ]

    @pl.when(i == 0)
    def _prelude():
        for s in range(_NS - 1):
            for cp in _copies(s, s):
                cp.start()
        z = jnp.dot(x_ref[...], W1_ref[...],
                    preferred_element_type=jnp.float32) + b_ref[...]
        z_ref[...] = z
        support_ref[...] = jnp.dot(z, W2_ref[...],
                                   preferred_element_type=jnp.float32)
        cl = cl_ref[...]
        c_norm = jnp.sqrt(jnp.sum(cl * cl, axis=1, keepdims=True))
        cln_ref[...] = cl / jnp.maximum(c_norm, 1e-8)

    slot = jax.lax.rem(i, _NS)
    for cp in _copies(i, slot):
        cp.wait()

    nxt = i + _NS - 1

    @pl.when(nxt < _NCHUNK)
    def _issue():
        for cp in _copies(nxt, jax.lax.rem(nxt, _NS)):
            cp.start()

    h1_ref[...] = jnp.tanh(
        jnp.dot(bufs[slot], support_ref[...],
                preferred_element_type=jnp.float32))

    z_blk = z_ref[pl.ds(i * _BM, _BM), :]
    num = jax.lax.dot_general(z_blk, cln_ref[...], (((1,), (1,)), ((), ())),
                              preferred_element_type=jnp.float32)
    z_norm = jnp.sqrt(jnp.sum(z_blk * z_blk, axis=1, keepdims=True))
    e = jnp.exp(num / jnp.maximum(z_norm, 1e-8))
    sim_ref[...] = e / jnp.sum(e, axis=1, keepdims=True)


def kernel(seq1, adj, W_ae1, b_ae1, W_gcn, cluster):
    x = seq1[0]
    A = adj[0]
    b2 = b_ae1.reshape(1, _D)
    grid = (_NCHUNK,)
    h1, sim = pl.pallas_call(
        _fused,
        grid=grid,
        in_specs=[
            pl.BlockSpec((_N, _D), lambda i: (0, 0)),
            pl.BlockSpec(memory_space=pl.ANY),
            pl.BlockSpec((_D, _D), lambda i: (0, 0)),
            pl.BlockSpec((1, _D), lambda i: (0, 0)),
            pl.BlockSpec((_D, _D), lambda i: (0, 0)),
            pl.BlockSpec((_K, _D), lambda i: (0, 0)),
        ],
        out_specs=[
            pl.BlockSpec((_BM, _D), lambda i: (i, 0)),
            pl.BlockSpec((_BM, _K), lambda i: (i, 0)),
        ],
        out_shape=[
            jax.ShapeDtypeStruct((_N, _D), jnp.float32),
            jax.ShapeDtypeStruct((_N, _K), jnp.float32),
        ],
        scratch_shapes=[
            pltpu.VMEM((_NS, _BM, _N), jnp.float32),
            pltpu.VMEM((_N, _D), jnp.float32),
            pltpu.VMEM((_N, _D), jnp.float32),
            pltpu.VMEM((_K, _D), jnp.float32),
            pltpu.SemaphoreType.DMA((_NS, 2)),
        ],
        compiler_params=pltpu.CompilerParams(
            dimension_semantics=("arbitrary",),
        ),
    )(x, A, W_ae1, b2, W_gcn, cluster)
    return (h1, sim)
